# baseline (speedup 1.0000x reference)
import functools
import jax
import jax.numpy as jnp
from jax import lax
from jax.experimental import pallas as pl
from jax.experimental.pallas import tpu as pltpu
from jax.experimental.pallas import tpu_sc as plsc

_N1 = 10000
_N2 = 20000
_E1 = 320000
_E2 = 640000
_D_IN = 128
_H = 64
_C = 125
_NEG = 0.01
_HTOT = 2 * _N1 + 2 * _N2


def _leaky(x):
    return jnp.where(x >= 0.0, x, x * _NEG)


def _make_hist():
    mesh = plsc.VectorSubcoreMesh(core_axis_name="c", subcore_axis_name="s")
    epts = [_E1 // 32, _E1 // 32, _E2 // 32, _E2 // 32]
    offs = [0, _N1, 2 * _N1, 2 * _N1 + _N2]

    @functools.partial(
        pl.kernel,
        out_type=jax.ShapeDtypeStruct((32, _HTOT), jnp.float32),
        mesh=mesh,
        scratch_types=[
            pltpu.VMEM((_HTOT,), jnp.float32),
            pltpu.VMEM((_E2 // 32,), jnp.int32),
        ],
        compiler_params=pltpu.CompilerParams(needs_layout_passes=False,
                                             use_tc_tiling_on_sc=False),
    )
    def hist_kernel(s1, d1, s2, d2, out, hist, ibuf):
        cid = lax.axis_index("c")
        sid = lax.axis_index("s")
        wid = cid * 16 + sid
        zero16 = jnp.zeros((16,), jnp.float32)

        def zloop(j, carry):
            hist[pl.ds(j * 16, 16)] = zero16
            return carry

        lax.fori_loop(0, _HTOT // 16, zloop, 0)

        one16 = jnp.ones((16,), jnp.float32)
        for arr, ept, off in zip([s1, d1, s2, d2], epts, offs):
            pltpu.sync_copy(arr.at[pl.ds(wid * ept, ept)], ibuf.at[pl.ds(0, ept)])

            def body(j, carry, _off=off):
                for u in range(5):
                    v = ibuf[pl.ds(j * 80 + u * 16, 16)] + _off
                    plsc.addupdate_scatter(hist, [v], one16)
                return carry

            lax.fori_loop(0, ept // 80, body, 0)

        pltpu.sync_copy(hist, out.at[wid])

    return hist_kernel


def _make_edge_pass(N, E):
    NCH = E // _C
    NCH_T = NCH // 32
    GB = 40
    NG = NCH_T // GB
    NB = 5
    WT = 10
    ROWS_W = N // WT
    ZB = 40
    NZ = ROWS_W // ZB
    mesh = plsc.VectorSubcoreMesh(core_axis_name="c", subcore_axis_name="s")

    @functools.partial(
        pl.kernel,
        out_type=jax.ShapeDtypeStruct((2, N, _H), jnp.float32),
        mesh=mesh,
        scratch_types=[
            pltpu.VMEM_SHARED((N, _H), jnp.float32),
            pltpu.VMEM((GB, _C), jnp.int32),
            pltpu.VMEM((GB, _C), jnp.int32),
            [pltpu.VMEM((_C, _H), jnp.float32)] * NB,
            [pltpu.SemaphoreType.DMA] * NB,
            [pltpu.SemaphoreType.DMA] * NB,
        ],
        compiler_params=pltpu.CompilerParams(needs_layout_passes=False,
                                             use_tc_tiling_on_sc=False),
    )
    def ep(table, src3d, dst3d, out, acc, sidx, didx, rows, gsem, ssem):
        cid = lax.axis_index("c")
        sid = lax.axis_index("s")
        wid = cid * 16 + sid
        zero16 = jnp.zeros((16,), jnp.float32)

        def zrow(r, carry):
            for c4 in range(_H // 16):
                rows[0][r, pl.ds(c4 * 16, 16)] = zero16
            return carry

        lax.fori_loop(0, ZB, zrow, 0)

        rbase = sid * ROWS_W

        @pl.when(sid < WT)
        def _():
            for k in range(NZ):
                pltpu.sync_copy(rows[0].at[pl.ds(0, ZB)],
                                acc.at[pl.ds(rbase + k * ZB, ZB)])

        plsc.subcore_barrier()

        def gissue(i, b):
            pltpu.async_copy(table.at[sidx.at[i]], rows[b], gsem[b])

        def gwait(i, b):
            pltpu.make_async_copy(table.at[sidx.at[i]], rows[b],
                                  gsem[b]).wait()

        def sissue(i, b):
            pltpu.async_copy(rows[b], acc.at[didx.at[i]], ssem[b], add=True)

        def swait(i, b):
            pltpu.make_async_copy(rows[b], acc.at[didx.at[i]],
                                  ssem[b]).wait()

        for g in range(NG):
            pltpu.sync_copy(src3d.at[wid, pl.ds(g * GB, GB)], sidx)
            pltpu.sync_copy(dst3d.at[wid, pl.ds(g * GB, GB)], didx)
            for b in range(3):
                gissue(b, b)

            def blk_body(blk, carry):
                for j in range(NB):
                    i = blk * NB + j
                    bf = (j + 3) % NB

                    @pl.when(i >= 2)
                    def _():
                        swait(i - 2, bf)

                    @pl.when(i + 3 < GB)
                    def _():
                        gissue(i + 3, bf)

                    gwait(i, j)
                    sissue(i, j)
                return carry

            lax.fori_loop(0, GB // NB, blk_body, 0)
            swait(GB - 2, (GB - 2) % NB)
            swait(GB - 1, (GB - 1) % NB)

        plsc.subcore_barrier()

        @pl.when(sid < WT)
        def _():
            pltpu.sync_copy(acc.at[pl.ds(rbase, ROWS_W)],
                            out.at[cid, pl.ds(rbase, ROWS_W)])

    return ep


_BM = 1000


def _norm_cols(deg_blk):
    s = jnp.sum(deg_blk[0], axis=0)[:, None]
    return lax.rsqrt(jnp.maximum(s, 1.0))


def _table1_tc(x, W1, deg):
    def body(x_ref, w_ref, d_ref, o_ref):
        ns1 = _norm_cols(d_ref[...])
        o_ref[...] = jnp.dot(x_ref[...], w_ref[...],
                             preferred_element_type=jnp.float32) * ns1

    return pl.pallas_call(
        body,
        grid=(_N1 // _BM,),
        in_specs=[
            pl.BlockSpec((_BM, _D_IN), lambda r: (r, 0)),
            pl.BlockSpec((_D_IN, _H), lambda r: (0, 0)),
            pl.BlockSpec((1, 32, _BM), lambda r: (r, 0, 0)),
        ],
        out_specs=pl.BlockSpec((_BM, _H), lambda r: (r, 0)),
        out_shape=jax.ShapeDtypeStruct((_N1, _H), jnp.float32),
    )(x, W1, deg)


def _table2_tc(p0, p1, b1, W2, deg):
    NB = _N1 // _BM

    def body(p0_ref, p1_ref, b_ref, w_ref, dn_ref, ds_ref, o_ref):
        nd1 = _norm_cols(dn_ref[...])
        ns2 = _norm_cols(ds_ref[...])
        h = (p0_ref[...] + p1_ref[...]) * nd1 + b_ref[...]
        h = _leaky(h)
        o_ref[...] = jnp.dot(h, w_ref[...],
                             preferred_element_type=jnp.float32) * ns2

    return pl.pallas_call(
        body,
        grid=(2, NB),
        in_specs=[
            pl.BlockSpec((_BM, _H), lambda c, r: (r, 0)),
            pl.BlockSpec((_BM, _H), lambda c, r: (r, 0)),
            pl.BlockSpec((1, _H), lambda c, r: (0, 0)),
            pl.BlockSpec((_H, _H), lambda c, r: (0, 0)),
            pl.BlockSpec((1, 32, _BM), lambda c, r: (NB + r, 0, 0)),
            pl.BlockSpec((1, 32, _BM), lambda c, r: (2 * NB + c * NB + r, 0, 0)),
        ],
        out_specs=pl.BlockSpec((_BM, _H), lambda c, r: (c * NB + r, 0)),
        out_shape=jax.ShapeDtypeStruct((_N2, _H), jnp.float32),
    )(p0, p1, b1, W2, deg, deg)


def _final_tc(p0, p1, b2, deg):
    NB1 = _N1 // _BM

    def body(p0_ref, p1_ref, b_ref, d_ref, o_ref):
        nd2 = _norm_cols(d_ref[...])
        o_ref[...] = _leaky((p0_ref[...] + p1_ref[...]) * nd2 + b_ref[...])

    return pl.pallas_call(
        body,
        grid=(_N2 // _BM,),
        in_specs=[
            pl.BlockSpec((_BM, _H), lambda r: (r, 0)),
            pl.BlockSpec((_BM, _H), lambda r: (r, 0)),
            pl.BlockSpec((1, _H), lambda r: (0, 0)),
            pl.BlockSpec((1, 32, _BM), lambda r: (4 * NB1 + r, 0, 0)),
        ],
        out_specs=pl.BlockSpec((_BM, _H), lambda r: (r, 0)),
        out_shape=jax.ShapeDtypeStruct((_N2, _H), jnp.float32),
    )(p0, p1, b2, deg)


_hist = _make_hist()
_edge1 = _make_edge_pass(_N1, _E1)
_edge2 = _make_edge_pass(_N2, _E2)


def kernel(node_features, edge_index1, edge_index2, W1, b1, W2, b2):
    src1 = edge_index1[0]
    dst1 = edge_index1[1]
    src2 = edge_index2[0]
    dst2 = edge_index2[1]

    deg = _hist(src1, dst1, src2, dst2)
    deg = deg.reshape(32, _HTOT // _BM, _BM).transpose(1, 0, 2)

    table1 = _table1_tc(node_features, W1, deg)
    p1 = _edge1(table1,
                src1.reshape(32, _E1 // _C // 32, _C),
                dst1.reshape(32, _E1 // _C // 32, _C))
    table2 = _table2_tc(p1[0], p1[1], b1[None, :], W2, deg)
    p2 = _edge2(table2,
                src2.reshape(32, _E2 // _C // 32, _C),
                dst2.reshape(32, _E2 // _C // 32, _C))
    return _final_tc(p2[0], p2[1], b2[None, :], deg)


# speedup vs baseline: 1.0400x; 1.0400x over previous
import functools
import jax
import jax.numpy as jnp
from jax import lax
from jax.experimental import pallas as pl
from jax.experimental.pallas import tpu as pltpu
from jax.experimental.pallas import tpu_sc as plsc

_N1 = 10000
_N2 = 20000
_E1 = 320000
_E2 = 640000
_D_IN = 128
_H = 64
_C = 125
_NEG = 0.01
_HTOT = 2 * _N1 + 2 * _N2


def _leaky(x):
    return jnp.where(x >= 0.0, x, x * _NEG)


def _make_hist():
    mesh = plsc.VectorSubcoreMesh(core_axis_name="c", subcore_axis_name="s")
    epts = [_E1 // 32, _E1 // 32, _E2 // 32, _E2 // 32]
    offs = [0, _N1, 2 * _N1, 2 * _N1 + _N2]

    @functools.partial(
        pl.kernel,
        out_type=jax.ShapeDtypeStruct((_HTOT // 1000, 32, 1000), jnp.float32),
        mesh=mesh,
        scratch_types=[
            pltpu.VMEM((_HTOT,), jnp.float32),
            pltpu.VMEM((_E2 // 32,), jnp.int32),
            pltpu.SemaphoreType.DMA,
        ],
        compiler_params=pltpu.CompilerParams(needs_layout_passes=False,
                                             use_tc_tiling_on_sc=False),
    )
    def hist_kernel(s1, d1, s2, d2, out, hist, ibuf, osem):
        cid = lax.axis_index("c")
        sid = lax.axis_index("s")
        wid = cid * 16 + sid
        zero16 = jnp.zeros((16,), jnp.float32)

        def zloop(j, carry):
            hist[pl.ds(j * 16, 16)] = zero16
            return carry

        lax.fori_loop(0, _HTOT // 16, zloop, 0)

        one16 = jnp.ones((16,), jnp.float32)
        for arr, ept, off in zip([s1, d1, s2, d2], epts, offs):
            pltpu.sync_copy(arr.at[pl.ds(wid * ept, ept)], ibuf.at[pl.ds(0, ept)])

            def body(j, carry, _off=off):
                for u in range(5):
                    v = ibuf[pl.ds(j * 80 + u * 16, 16)] + _off
                    plsc.addupdate_scatter(hist, [v], one16)
                return carry

            lax.fori_loop(0, ept // 80, body, 0)

        for k in range(_HTOT // 1000):
            pltpu.async_copy(hist.at[pl.ds(k * 1000, 1000)],
                             out.at[k, wid], osem)
        for k in range(_HTOT // 1000):
            pltpu.make_async_copy(hist.at[pl.ds(k * 1000, 1000)],
                                  out.at[k, wid], osem).wait()

    return hist_kernel


def _make_edge_pass(N, E):
    NCH = E // _C
    NCH_T = NCH // 32
    GB = 40
    NG = NCH_T // GB
    NB = 5
    WT = 10
    ROWS_W = N // WT
    ZB = 40
    NZ = ROWS_W // ZB
    mesh = plsc.VectorSubcoreMesh(core_axis_name="c", subcore_axis_name="s")

    @functools.partial(
        pl.kernel,
        out_type=jax.ShapeDtypeStruct((2, N, _H), jnp.float32),
        mesh=mesh,
        scratch_types=[
            pltpu.VMEM_SHARED((N, _H), jnp.float32),
            pltpu.VMEM((GB, _C), jnp.int32),
            pltpu.VMEM((GB, _C), jnp.int32),
            [pltpu.VMEM((_C, _H), jnp.float32)] * NB,
            [pltpu.SemaphoreType.DMA] * NB,
            [pltpu.SemaphoreType.DMA] * NB,
        ],
        compiler_params=pltpu.CompilerParams(needs_layout_passes=False,
                                             use_tc_tiling_on_sc=False),
    )
    def ep(table, src3d, dst3d, out, acc, sidx, didx, rows, gsem, ssem):
        cid = lax.axis_index("c")
        sid = lax.axis_index("s")
        wid = cid * 16 + sid
        zero16 = jnp.zeros((16,), jnp.float32)

        def zrow(r, carry):
            for c4 in range(_H // 16):
                rows[0][r, pl.ds(c4 * 16, 16)] = zero16
            return carry

        lax.fori_loop(0, ZB, zrow, 0)

        rbase = sid * ROWS_W

        @pl.when(sid < WT)
        def _():
            for k in range(NZ):
                pltpu.sync_copy(rows[0].at[pl.ds(0, ZB)],
                                acc.at[pl.ds(rbase + k * ZB, ZB)])

        plsc.subcore_barrier()

        def gissue(i, b):
            pltpu.async_copy(table.at[sidx.at[i]], rows[b], gsem[b])

        def gwait(i, b):
            pltpu.make_async_copy(table.at[sidx.at[i]], rows[b],
                                  gsem[b]).wait()

        def sissue(i, b):
            pltpu.async_copy(rows[b], acc.at[didx.at[i]], ssem[b], add=True)

        def swait(i, b):
            pltpu.make_async_copy(rows[b], acc.at[didx.at[i]],
                                  ssem[b]).wait()

        for g in range(NG):
            pltpu.sync_copy(src3d.at[wid, pl.ds(g * GB, GB)], sidx)
            pltpu.sync_copy(dst3d.at[wid, pl.ds(g * GB, GB)], didx)
            for b in range(3):
                gissue(b, b)

            def blk_body(blk, carry):
                for j in range(NB):
                    i = blk * NB + j
                    bf = (j + 3) % NB

                    @pl.when(i >= 2)
                    def _():
                        swait(i - 2, bf)

                    @pl.when(i + 3 < GB)
                    def _():
                        gissue(i + 3, bf)

                    gwait(i, j)
                    sissue(i, j)
                return carry

            lax.fori_loop(0, GB // NB, blk_body, 0)
            swait(GB - 2, (GB - 2) % NB)
            swait(GB - 1, (GB - 1) % NB)

        plsc.subcore_barrier()

        @pl.when(sid < WT)
        def _():
            pltpu.sync_copy(acc.at[pl.ds(rbase, ROWS_W)],
                            out.at[cid, pl.ds(rbase, ROWS_W)])

    return ep


_BM = 1000


def _norm_cols(deg_blk):
    s = jnp.sum(deg_blk[0], axis=0)[:, None]
    return lax.rsqrt(jnp.maximum(s, 1.0))


def _table1_tc(x, W1, deg):
    def body(x_ref, w_ref, d_ref, o_ref):
        ns1 = _norm_cols(d_ref[...])
        o_ref[...] = jnp.dot(x_ref[...], w_ref[...],
                             preferred_element_type=jnp.float32) * ns1

    return pl.pallas_call(
        body,
        grid=(_N1 // _BM,),
        in_specs=[
            pl.BlockSpec((_BM, _D_IN), lambda r: (r, 0)),
            pl.BlockSpec((_D_IN, _H), lambda r: (0, 0)),
            pl.BlockSpec((1, 32, _BM), lambda r: (r, 0, 0)),
        ],
        out_specs=pl.BlockSpec((_BM, _H), lambda r: (r, 0)),
        out_shape=jax.ShapeDtypeStruct((_N1, _H), jnp.float32),
    )(x, W1, deg)


def _table2_tc(p0, p1, b1, W2, deg):
    NB = _N1 // _BM

    def body(p0_ref, p1_ref, b_ref, w_ref, dn_ref, ds_ref, o_ref):
        nd1 = _norm_cols(dn_ref[...])
        ns2 = _norm_cols(ds_ref[...])
        h = (p0_ref[...] + p1_ref[...]) * nd1 + b_ref[...]
        h = _leaky(h)
        o_ref[...] = jnp.dot(h, w_ref[...],
                             preferred_element_type=jnp.float32) * ns2

    return pl.pallas_call(
        body,
        grid=(2, NB),
        in_specs=[
            pl.BlockSpec((_BM, _H), lambda c, r: (r, 0)),
            pl.BlockSpec((_BM, _H), lambda c, r: (r, 0)),
            pl.BlockSpec((1, _H), lambda c, r: (0, 0)),
            pl.BlockSpec((_H, _H), lambda c, r: (0, 0)),
            pl.BlockSpec((1, 32, _BM), lambda c, r: (NB + r, 0, 0)),
            pl.BlockSpec((1, 32, _BM), lambda c, r: (2 * NB + c * NB + r, 0, 0)),
        ],
        out_specs=pl.BlockSpec((_BM, _H), lambda c, r: (c * NB + r, 0)),
        out_shape=jax.ShapeDtypeStruct((_N2, _H), jnp.float32),
    )(p0, p1, b1, W2, deg, deg)


def _final_tc(p0, p1, b2, deg):
    NB1 = _N1 // _BM

    def body(p0_ref, p1_ref, b_ref, d_ref, o_ref):
        nd2 = _norm_cols(d_ref[...])
        o_ref[...] = _leaky((p0_ref[...] + p1_ref[...]) * nd2 + b_ref[...])

    return pl.pallas_call(
        body,
        grid=(_N2 // _BM,),
        in_specs=[
            pl.BlockSpec((_BM, _H), lambda r: (r, 0)),
            pl.BlockSpec((_BM, _H), lambda r: (r, 0)),
            pl.BlockSpec((1, _H), lambda r: (0, 0)),
            pl.BlockSpec((1, 32, _BM), lambda r: (4 * NB1 + r, 0, 0)),
        ],
        out_specs=pl.BlockSpec((_BM, _H), lambda r: (r, 0)),
        out_shape=jax.ShapeDtypeStruct((_N2, _H), jnp.float32),
    )(p0, p1, b2, deg)


_hist = _make_hist()
_edge1 = _make_edge_pass(_N1, _E1)
_edge2 = _make_edge_pass(_N2, _E2)


def kernel(node_features, edge_index1, edge_index2, W1, b1, W2, b2):
    src1 = edge_index1[0]
    dst1 = edge_index1[1]
    src2 = edge_index2[0]
    dst2 = edge_index2[1]

    deg = _hist(src1, dst1, src2, dst2)

    table1 = _table1_tc(node_features, W1, deg)
    p1 = _edge1(table1,
                src1.reshape(32, _E1 // _C // 32, _C),
                dst1.reshape(32, _E1 // _C // 32, _C))
    table2 = _table2_tc(p1[0], p1[1], b1[None, :], W2, deg)
    p2 = _edge2(table2,
                src2.reshape(32, _E2 // _C // 32, _C),
                dst2.reshape(32, _E2 // _C // 32, _C))
    return _final_tc(p2[0], p2[1], b2[None, :], deg)


# speedup vs baseline: 1.2227x; 1.1756x over previous
import functools
import jax
import jax.numpy as jnp
from jax import lax
from jax.experimental import pallas as pl
from jax.experimental.pallas import tpu as pltpu
from jax.experimental.pallas import tpu_sc as plsc

_N1 = 10000
_N2 = 20000
_E1 = 320000
_E2 = 640000
_D_IN = 128
_H = 64
_C = 125
_NEG = 0.01
_HTOT = 2 * _N1 + 2 * _N2


def _leaky(x):
    return jnp.where(x >= 0.0, x, x * _NEG)


def _make_hist():
    mesh = plsc.VectorSubcoreMesh(core_axis_name="c", subcore_axis_name="s")
    epts = [_E1 // 32, _E1 // 32, _E2 // 32, _E2 // 32]
    offs = [0, _N1, 2 * _N1, 2 * _N1 + _N2]

    @functools.partial(
        pl.kernel,
        out_type=jax.ShapeDtypeStruct((_HTOT // 1000, 32, 1000), jnp.float32),
        mesh=mesh,
        scratch_types=[
            pltpu.VMEM((_HTOT,), jnp.float32),
            pltpu.VMEM((_E2 // 32,), jnp.int32),
            pltpu.SemaphoreType.DMA,
        ],
        compiler_params=pltpu.CompilerParams(needs_layout_passes=False,
                                             use_tc_tiling_on_sc=False),
    )
    def hist_kernel(e1, e2, out, hist, ibuf, osem):
        cid = lax.axis_index("c")
        sid = lax.axis_index("s")
        wid = cid * 16 + sid
        zero16 = jnp.zeros((16,), jnp.float32)

        def zloop(j, carry):
            hist[pl.ds(j * 16, 16)] = zero16
            return carry

        lax.fori_loop(0, _HTOT // 16, zloop, 0)

        one16 = jnp.ones((16,), jnp.float32)
        srcs = [(e1, 0), (e1, 1), (e2, 0), (e2, 1)]
        for (arr, row), ept, off in zip(srcs, epts, offs):
            pltpu.sync_copy(arr.at[row, wid], ibuf.at[pl.ds(0, ept)])

            def body(j, carry, _off=off):
                for u in range(5):
                    v = ibuf[pl.ds(j * 80 + u * 16, 16)] + _off
                    plsc.addupdate_scatter(hist, [v], one16)
                return carry

            lax.fori_loop(0, ept // 80, body, 0)

        for k in range(_HTOT // 1000):
            pltpu.async_copy(hist.at[pl.ds(k * 1000, 1000)],
                             out.at[k, wid], osem)
        for k in range(_HTOT // 1000):
            pltpu.make_async_copy(hist.at[pl.ds(k * 1000, 1000)],
                                  out.at[k, wid], osem).wait()

    return hist_kernel


def _make_edge_pass(N, E):
    NCH = E // _C
    NCH_T = NCH // 32
    GB = 40
    NG = NCH_T // GB
    NB = 5
    WT = 10
    ROWS_W = N // WT
    ZB = 40
    NZ = ROWS_W // ZB
    mesh = plsc.VectorSubcoreMesh(core_axis_name="c", subcore_axis_name="s")

    @functools.partial(
        pl.kernel,
        out_type=jax.ShapeDtypeStruct((2, N, _H), jnp.float32),
        mesh=mesh,
        scratch_types=[
            pltpu.VMEM_SHARED((N, _H), jnp.float32),
            pltpu.VMEM((GB, _C), jnp.int32),
            pltpu.VMEM((GB, _C), jnp.int32),
            [pltpu.VMEM((_C, _H), jnp.float32)] * NB,
            [pltpu.SemaphoreType.DMA] * NB,
            [pltpu.SemaphoreType.DMA] * NB,
        ],
        compiler_params=pltpu.CompilerParams(needs_layout_passes=False,
                                             use_tc_tiling_on_sc=False),
    )
    def ep(table, eidx, out, acc, sidx, didx, rows, gsem, ssem):
        cid = lax.axis_index("c")
        sid = lax.axis_index("s")
        wid = cid * 16 + sid
        zero16 = jnp.zeros((16,), jnp.float32)

        def zrow(r, carry):
            for c4 in range(_H // 16):
                rows[0][r, pl.ds(c4 * 16, 16)] = zero16
            return carry

        lax.fori_loop(0, ZB, zrow, 0)

        rbase = sid * ROWS_W

        @pl.when(sid < WT)
        def _():
            for k in range(NZ):
                pltpu.sync_copy(rows[0].at[pl.ds(0, ZB)],
                                acc.at[pl.ds(rbase + k * ZB, ZB)])

        plsc.subcore_barrier()

        def gissue(i, b):
            pltpu.async_copy(table.at[sidx.at[i]], rows[b], gsem[b])

        def gwait(i, b):
            pltpu.make_async_copy(table.at[sidx.at[i]], rows[b],
                                  gsem[b]).wait()

        def sissue(i, b):
            pltpu.async_copy(rows[b], acc.at[didx.at[i]], ssem[b], add=True)

        def swait(i, b):
            pltpu.make_async_copy(rows[b], acc.at[didx.at[i]],
                                  ssem[b]).wait()

        for g in range(NG):
            pltpu.sync_copy(eidx.at[0, wid, pl.ds(g * GB, GB)], sidx)
            pltpu.sync_copy(eidx.at[1, wid, pl.ds(g * GB, GB)], didx)
            for b in range(3):
                gissue(b, b)

            def blk_body(blk, carry):
                for j in range(NB):
                    i = blk * NB + j
                    bf = (j + 3) % NB

                    @pl.when(i >= 2)
                    def _():
                        swait(i - 2, bf)

                    @pl.when(i + 3 < GB)
                    def _():
                        gissue(i + 3, bf)

                    gwait(i, j)
                    sissue(i, j)
                return carry

            lax.fori_loop(0, GB // NB, blk_body, 0)
            swait(GB - 2, (GB - 2) % NB)
            swait(GB - 1, (GB - 1) % NB)

        plsc.subcore_barrier()

        @pl.when(sid < WT)
        def _():
            pltpu.sync_copy(acc.at[pl.ds(rbase, ROWS_W)],
                            out.at[cid, pl.ds(rbase, ROWS_W)])

    return ep


_BM = 1000


def _norm_cols(deg_blk):
    s = jnp.sum(deg_blk[0], axis=0)[:, None]
    return lax.rsqrt(jnp.maximum(s, 1.0))


def _table1_tc(x, W1, deg):
    def body(x_ref, w_ref, d_ref, o_ref):
        ns1 = _norm_cols(d_ref[...])
        o_ref[...] = jnp.dot(x_ref[...], w_ref[...],
                             preferred_element_type=jnp.float32) * ns1

    return pl.pallas_call(
        body,
        grid=(_N1 // _BM,),
        in_specs=[
            pl.BlockSpec((_BM, _D_IN), lambda r: (r, 0)),
            pl.BlockSpec((_D_IN, _H), lambda r: (0, 0)),
            pl.BlockSpec((1, 32, _BM), lambda r: (r, 0, 0)),
        ],
        out_specs=pl.BlockSpec((_BM, _H), lambda r: (r, 0)),
        out_shape=jax.ShapeDtypeStruct((_N1, _H), jnp.float32),
    )(x, W1, deg)


def _table2_tc(p0, p1, b1, W2, deg):
    NB = _N1 // _BM

    def body(p0_ref, p1_ref, b_ref, w_ref, dn_ref, ds_ref, o_ref):
        nd1 = _norm_cols(dn_ref[...])
        ns2 = _norm_cols(ds_ref[...])
        h = (p0_ref[0] + p1_ref[0]) * nd1 + b_ref[...]
        h = _leaky(h)
        o_ref[...] = jnp.dot(h, w_ref[...],
                             preferred_element_type=jnp.float32) * ns2

    return pl.pallas_call(
        body,
        grid=(2, NB),
        in_specs=[
            pl.BlockSpec((1, _BM, _H), lambda c, r: (0, r, 0)),
            pl.BlockSpec((1, _BM, _H), lambda c, r: (1, r, 0)),
            pl.BlockSpec((1, _H), lambda c, r: (0, 0)),
            pl.BlockSpec((_H, _H), lambda c, r: (0, 0)),
            pl.BlockSpec((1, 32, _BM), lambda c, r: (NB + r, 0, 0)),
            pl.BlockSpec((1, 32, _BM), lambda c, r: (2 * NB + c * NB + r, 0, 0)),
        ],
        out_specs=pl.BlockSpec((_BM, _H), lambda c, r: (c * NB + r, 0)),
        out_shape=jax.ShapeDtypeStruct((_N2, _H), jnp.float32),
    )(p0, p1, b1, W2, deg, deg)


def _final_tc(p0, p1, b2, deg):
    NB1 = _N1 // _BM

    def body(p0_ref, p1_ref, b_ref, d_ref, o_ref):
        nd2 = _norm_cols(d_ref[...])
        o_ref[...] = _leaky((p0_ref[0] + p1_ref[0]) * nd2 + b_ref[...])

    return pl.pallas_call(
        body,
        grid=(_N2 // _BM,),
        in_specs=[
            pl.BlockSpec((1, _BM, _H), lambda r: (0, r, 0)),
            pl.BlockSpec((1, _BM, _H), lambda r: (1, r, 0)),
            pl.BlockSpec((1, _H), lambda r: (0, 0)),
            pl.BlockSpec((1, 32, _BM), lambda r: (4 * NB1 + r, 0, 0)),
        ],
        out_specs=pl.BlockSpec((_BM, _H), lambda r: (r, 0)),
        out_shape=jax.ShapeDtypeStruct((_N2, _H), jnp.float32),
    )(p0, p1, b2, deg)


_hist = _make_hist()
_edge1 = _make_edge_pass(_N1, _E1)
_edge2 = _make_edge_pass(_N2, _E2)


def kernel(node_features, edge_index1, edge_index2, W1, b1, W2, b2):
    e1h = edge_index1.reshape(2, 32, _E1 // 32)
    e2h = edge_index2.reshape(2, 32, _E2 // 32)
    e1c = edge_index1.reshape(2, 32, _E1 // _C // 32, _C)
    e2c = edge_index2.reshape(2, 32, _E2 // _C // 32, _C)

    deg = _hist(e1h, e2h)

    table1 = _table1_tc(node_features, W1, deg)
    p1 = _edge1(table1, e1c)
    table2 = _table2_tc(p1, p1, b1[None, :], W2, deg)
    p2 = _edge2(table2, e2c)
    return _final_tc(p2, p2, b2[None, :], deg)


# speedup vs baseline: 1.2701x; 1.0388x over previous
import functools
import jax
import jax.numpy as jnp
from jax import lax
from jax.experimental import pallas as pl
from jax.experimental.pallas import tpu as pltpu
from jax.experimental.pallas import tpu_sc as plsc

_N1 = 10000
_N2 = 20000
_E1 = 320000
_E2 = 640000
_D_IN = 128
_H = 64
_C = 125
_NEG = 0.01
_HTOT = 2 * _N1 + 2 * _N2


def _leaky(x):
    return jnp.where(x >= 0.0, x, x * _NEG)


def _make_hist():
    mesh = plsc.VectorSubcoreMesh(core_axis_name="c", subcore_axis_name="s")
    epts = [_E1 // 32, _E1 // 32, _E2 // 32, _E2 // 32]
    offs = [0, _N1, 2 * _N1, 2 * _N1 + _N2]

    @functools.partial(
        pl.kernel,
        out_type=jax.ShapeDtypeStruct((_HTOT // 2000, 32, 2000), jnp.float32),
        mesh=mesh,
        scratch_types=[
            pltpu.VMEM((_HTOT,), jnp.float32),
            pltpu.VMEM((_E2 // 32,), jnp.int32),
            pltpu.SemaphoreType.DMA,
        ],
        compiler_params=pltpu.CompilerParams(needs_layout_passes=False,
                                             use_tc_tiling_on_sc=False),
    )
    def hist_kernel(e1, e2, out, hist, ibuf, osem):
        cid = lax.axis_index("c")
        sid = lax.axis_index("s")
        wid = cid * 16 + sid
        zero16 = jnp.zeros((16,), jnp.float32)

        def zloop(j, carry):
            hist[pl.ds(j * 16, 16)] = zero16
            return carry

        lax.fori_loop(0, _HTOT // 16, zloop, 0)

        one16 = jnp.ones((16,), jnp.float32)
        srcs = [(e1, 0), (e1, 1), (e2, 0), (e2, 1)]
        for (arr, row), ept, off in zip(srcs, epts, offs):
            pltpu.sync_copy(arr.at[row, wid], ibuf.at[pl.ds(0, ept)])

            def body(j, carry, _off=off):
                for u in range(5):
                    v = ibuf[pl.ds(j * 80 + u * 16, 16)] + _off
                    plsc.addupdate_scatter(hist, [v], one16)
                return carry

            lax.fori_loop(0, ept // 80, body, 0)

        for k in range(_HTOT // 2000):
            pltpu.async_copy(hist.at[pl.ds(k * 2000, 2000)],
                             out.at[k, wid], osem)
        for k in range(_HTOT // 2000):
            pltpu.make_async_copy(hist.at[pl.ds(k * 2000, 2000)],
                                  out.at[k, wid], osem).wait()

    return hist_kernel


def _make_edge_pass(N, E):
    NCH = E // _C
    NCH_T = NCH // 32
    GB = 40
    NG = NCH_T // GB
    NB = 5
    WT = 10
    ROWS_W = N // WT
    ZB = 40
    NZ = ROWS_W // ZB
    mesh = plsc.VectorSubcoreMesh(core_axis_name="c", subcore_axis_name="s")

    @functools.partial(
        pl.kernel,
        out_type=jax.ShapeDtypeStruct((2, N, _H), jnp.float32),
        mesh=mesh,
        scratch_types=[
            pltpu.VMEM_SHARED((N, _H), jnp.float32),
            pltpu.VMEM((GB, _C), jnp.int32),
            pltpu.VMEM((GB, _C), jnp.int32),
            [pltpu.VMEM((_C, _H), jnp.float32)] * NB,
            [pltpu.SemaphoreType.DMA] * NB,
            [pltpu.SemaphoreType.DMA] * NB,
        ],
        compiler_params=pltpu.CompilerParams(needs_layout_passes=False,
                                             use_tc_tiling_on_sc=False),
    )
    def ep(table, eidx, out, acc, sidx, didx, rows, gsem, ssem):
        cid = lax.axis_index("c")
        sid = lax.axis_index("s")
        wid = cid * 16 + sid
        zero16 = jnp.zeros((16,), jnp.float32)

        def zrow(r, carry):
            for c4 in range(_H // 16):
                rows[0][r, pl.ds(c4 * 16, 16)] = zero16
            return carry

        lax.fori_loop(0, ZB, zrow, 0)

        rbase = sid * ROWS_W

        @pl.when(sid < WT)
        def _():
            for k in range(NZ):
                pltpu.sync_copy(rows[0].at[pl.ds(0, ZB)],
                                acc.at[pl.ds(rbase + k * ZB, ZB)])

        plsc.subcore_barrier()

        def gissue(i, b):
            pltpu.async_copy(table.at[sidx.at[i]], rows[b], gsem[b])

        def gwait(i, b):
            pltpu.make_async_copy(table.at[sidx.at[i]], rows[b],
                                  gsem[b]).wait()

        def sissue(i, b):
            pltpu.async_copy(rows[b], acc.at[didx.at[i]], ssem[b], add=True)

        def swait(i, b):
            pltpu.make_async_copy(rows[b], acc.at[didx.at[i]],
                                  ssem[b]).wait()

        for g in range(NG):
            pltpu.sync_copy(eidx.at[0, wid, pl.ds(g * GB, GB)], sidx)
            pltpu.sync_copy(eidx.at[1, wid, pl.ds(g * GB, GB)], didx)
            for b in range(3):
                gissue(b, b)

            def blk_body(blk, carry):
                for j in range(NB):
                    i = blk * NB + j
                    bf = (j + 3) % NB

                    @pl.when(i >= 2)
                    def _():
                        swait(i - 2, bf)

                    @pl.when(i + 3 < GB)
                    def _():
                        gissue(i + 3, bf)

                    gwait(i, j)
                    sissue(i, j)
                return carry

            lax.fori_loop(0, GB // NB, blk_body, 0)
            swait(GB - 2, (GB - 2) % NB)
            swait(GB - 1, (GB - 1) % NB)

        plsc.subcore_barrier()

        @pl.when(sid < WT)
        def _():
            pltpu.sync_copy(acc.at[pl.ds(rbase, ROWS_W)],
                            out.at[cid, pl.ds(rbase, ROWS_W)])

    return ep


_BM = 2000


def _norm_cols(deg_blk):
    s = jnp.sum(deg_blk[0], axis=0)[:, None]
    return lax.rsqrt(jnp.maximum(s, 1.0))


def _table1_tc(x, W1, deg):
    def body(x_ref, w_ref, d_ref, o_ref):
        ns1 = _norm_cols(d_ref[...])
        o_ref[...] = jnp.dot(x_ref[...], w_ref[...],
                             preferred_element_type=jnp.float32) * ns1

    return pl.pallas_call(
        body,
        grid=(_N1 // _BM,),
        in_specs=[
            pl.BlockSpec((_BM, _D_IN), lambda r: (r, 0)),
            pl.BlockSpec((_D_IN, _H), lambda r: (0, 0)),
            pl.BlockSpec((1, 32, _BM), lambda r: (r, 0, 0)),
        ],
        out_specs=pl.BlockSpec((_BM, _H), lambda r: (r, 0)),
        out_shape=jax.ShapeDtypeStruct((_N1, _H), jnp.float32),
    )(x, W1, deg)


def _table2_tc(p0, p1, b1, W2, deg):
    NB = _N1 // _BM

    def body(p0_ref, p1_ref, b_ref, w_ref, dn_ref, ds_ref, o_ref):
        nd1 = _norm_cols(dn_ref[...])
        ns2 = _norm_cols(ds_ref[...])
        h = (p0_ref[0] + p1_ref[0]) * nd1 + b_ref[...]
        h = _leaky(h)
        o_ref[...] = jnp.dot(h, w_ref[...],
                             preferred_element_type=jnp.float32) * ns2

    return pl.pallas_call(
        body,
        grid=(2, NB),
        in_specs=[
            pl.BlockSpec((1, _BM, _H), lambda c, r: (0, r, 0)),
            pl.BlockSpec((1, _BM, _H), lambda c, r: (1, r, 0)),
            pl.BlockSpec((1, _H), lambda c, r: (0, 0)),
            pl.BlockSpec((_H, _H), lambda c, r: (0, 0)),
            pl.BlockSpec((1, 32, _BM), lambda c, r: (NB + r, 0, 0)),
            pl.BlockSpec((1, 32, _BM), lambda c, r: (2 * NB + c * NB + r, 0, 0)),
        ],
        out_specs=pl.BlockSpec((_BM, _H), lambda c, r: (c * NB + r, 0)),
        out_shape=jax.ShapeDtypeStruct((_N2, _H), jnp.float32),
    )(p0, p1, b1, W2, deg, deg)


def _final_tc(p0, p1, b2, deg):
    NB1 = _N1 // _BM

    def body(p0_ref, p1_ref, b_ref, d_ref, o_ref):
        nd2 = _norm_cols(d_ref[...])
        o_ref[...] = _leaky((p0_ref[0] + p1_ref[0]) * nd2 + b_ref[...])

    return pl.pallas_call(
        body,
        grid=(_N2 // _BM,),
        in_specs=[
            pl.BlockSpec((1, _BM, _H), lambda r: (0, r, 0)),
            pl.BlockSpec((1, _BM, _H), lambda r: (1, r, 0)),
            pl.BlockSpec((1, _H), lambda r: (0, 0)),
            pl.BlockSpec((1, 32, _BM), lambda r: (4 * NB1 + r, 0, 0)),
        ],
        out_specs=pl.BlockSpec((_BM, _H), lambda r: (r, 0)),
        out_shape=jax.ShapeDtypeStruct((_N2, _H), jnp.float32),
    )(p0, p1, b2, deg)


_hist = _make_hist()
_edge1 = _make_edge_pass(_N1, _E1)
_edge2 = _make_edge_pass(_N2, _E2)


def kernel(node_features, edge_index1, edge_index2, W1, b1, W2, b2):
    e1h = edge_index1.reshape(2, 32, _E1 // 32)
    e2h = edge_index2.reshape(2, 32, _E2 // 32)
    e1c = edge_index1.reshape(2, 32, _E1 // _C // 32, _C)
    e2c = edge_index2.reshape(2, 32, _E2 // _C // 32, _C)

    deg = _hist(e1h, e2h)

    table1 = _table1_tc(node_features, W1, deg)
    p1 = _edge1(table1, e1c)
    table2 = _table2_tc(p1, p1, b1[None, :], W2, deg)
    p2 = _edge2(table2, e2c)
    return _final_tc(p2, p2, b2[None, :], deg)


# speedup vs baseline: 1.2760x; 1.0047x over previous
import functools
import jax
import jax.numpy as jnp
from jax import lax
from jax.experimental import pallas as pl
from jax.experimental.pallas import tpu as pltpu
from jax.experimental.pallas import tpu_sc as plsc

_N1 = 10000
_N2 = 20000
_E1 = 320000
_E2 = 640000
_D_IN = 128
_H = 64
_C = 125
_NEG = 0.01
_HTOT = 2 * _N1 + 2 * _N2


def _leaky(x):
    return jnp.where(x >= 0.0, x, x * _NEG)


def _make_hist():
    mesh = plsc.VectorSubcoreMesh(core_axis_name="c", subcore_axis_name="s")
    epts = [_E1 // 32, _E1 // 32, _E2 // 32, _E2 // 32]
    offs = [0, _N1, 2 * _N1, 2 * _N1 + _N2]

    @functools.partial(
        pl.kernel,
        out_type=jax.ShapeDtypeStruct((_HTOT // 2000, 32, 2048), jnp.float32),
        mesh=mesh,
        scratch_types=[
            pltpu.VMEM((_HTOT,), jnp.float32),
            pltpu.VMEM((_E2 // 32,), jnp.int32),
            pltpu.SemaphoreType.DMA,
        ],
        compiler_params=pltpu.CompilerParams(needs_layout_passes=False,
                                             use_tc_tiling_on_sc=False),
    )
    def hist_kernel(e1, e2, out, hist, ibuf, osem):
        cid = lax.axis_index("c")
        sid = lax.axis_index("s")
        wid = cid * 16 + sid
        zero16 = jnp.zeros((16,), jnp.float32)

        def zloop(j, carry):
            hist[pl.ds(j * 16, 16)] = zero16
            return carry

        lax.fori_loop(0, _HTOT // 16, zloop, 0)

        one16 = jnp.ones((16,), jnp.float32)
        srcs = [(e1, 0), (e1, 1), (e2, 0), (e2, 1)]
        for (arr, row), ept, off in zip(srcs, epts, offs):
            pltpu.sync_copy(arr.at[row, wid], ibuf.at[pl.ds(0, ept)])

            def body(j, carry, _off=off):
                for u in range(5):
                    v = ibuf[pl.ds(j * 80 + u * 16, 16)] + _off
                    plsc.addupdate_scatter(hist, [v], one16)
                return carry

            lax.fori_loop(0, ept // 80, body, 0)

        for k in range(_HTOT // 2000):
            pltpu.async_copy(hist.at[pl.ds(k * 2000, 2000)],
                             out.at[k, wid, pl.ds(0, 2000)], osem)
        for k in range(_HTOT // 2000):
            pltpu.make_async_copy(hist.at[pl.ds(k * 2000, 2000)],
                                  out.at[k, wid, pl.ds(0, 2000)], osem).wait()

    return hist_kernel


def _make_edge_pass(N, E):
    NCH = E // _C
    NCH_T = NCH // 32
    GB = 40
    NG = NCH_T // GB
    NB = 5
    WT = 10
    ROWS_W = N // WT
    ZB = 40
    NZ = ROWS_W // ZB
    mesh = plsc.VectorSubcoreMesh(core_axis_name="c", subcore_axis_name="s")

    @functools.partial(
        pl.kernel,
        out_type=jax.ShapeDtypeStruct((2, N, _H), jnp.float32),
        mesh=mesh,
        scratch_types=[
            pltpu.VMEM_SHARED((N, _H), jnp.float32),
            pltpu.VMEM((GB, _C), jnp.int32),
            pltpu.VMEM((GB, _C), jnp.int32),
            [pltpu.VMEM((_C, _H), jnp.float32)] * NB,
            [pltpu.SemaphoreType.DMA] * NB,
            [pltpu.SemaphoreType.DMA] * NB,
        ],
        compiler_params=pltpu.CompilerParams(needs_layout_passes=False,
                                             use_tc_tiling_on_sc=False),
    )
    def ep(table, eidx, out, acc, sidx, didx, rows, gsem, ssem):
        cid = lax.axis_index("c")
        sid = lax.axis_index("s")
        wid = cid * 16 + sid
        zero16 = jnp.zeros((16,), jnp.float32)

        def zrow(r, carry):
            for c4 in range(_H // 16):
                rows[0][r, pl.ds(c4 * 16, 16)] = zero16
            return carry

        lax.fori_loop(0, ZB, zrow, 0)

        rbase = sid * ROWS_W

        @pl.when(sid < WT)
        def _():
            for k in range(NZ):
                pltpu.sync_copy(rows[0].at[pl.ds(0, ZB)],
                                acc.at[pl.ds(rbase + k * ZB, ZB)])

        plsc.subcore_barrier()

        def gissue(i, b):
            pltpu.async_copy(table.at[sidx.at[i]], rows[b], gsem[b])

        def gwait(i, b):
            pltpu.make_async_copy(table.at[sidx.at[i]], rows[b],
                                  gsem[b]).wait()

        def sissue(i, b):
            pltpu.async_copy(rows[b], acc.at[didx.at[i]], ssem[b], add=True)

        def swait(i, b):
            pltpu.make_async_copy(rows[b], acc.at[didx.at[i]],
                                  ssem[b]).wait()

        for g in range(NG):
            pltpu.sync_copy(eidx.at[0, wid, pl.ds(g * GB, GB)], sidx)
            pltpu.sync_copy(eidx.at[1, wid, pl.ds(g * GB, GB)], didx)
            for b in range(3):
                gissue(b, b)

            def blk_body(blk, carry):
                for j in range(NB):
                    i = blk * NB + j
                    bf = (j + 3) % NB

                    @pl.when(i >= 2)
                    def _():
                        swait(i - 2, bf)

                    @pl.when(i + 3 < GB)
                    def _():
                        gissue(i + 3, bf)

                    gwait(i, j)
                    sissue(i, j)
                return carry

            lax.fori_loop(0, GB // NB, blk_body, 0)
            swait(GB - 2, (GB - 2) % NB)
            swait(GB - 1, (GB - 1) % NB)

        plsc.subcore_barrier()

        @pl.when(sid < WT)
        def _():
            pltpu.sync_copy(acc.at[pl.ds(rbase, ROWS_W)],
                            out.at[cid, pl.ds(rbase, ROWS_W)])

    return ep


_BM = 2000


def _norm_cols(deg_blk):
    s = jnp.sum(deg_blk[0], axis=0)[:_BM][:, None]
    return lax.rsqrt(jnp.maximum(s, 1.0))


def _table1_tc(x, W1, deg):
    def body(x_ref, w_ref, d_ref, o_ref):
        ns1 = _norm_cols(d_ref[...])
        o_ref[...] = jnp.dot(x_ref[...], w_ref[...],
                             preferred_element_type=jnp.float32) * ns1

    return pl.pallas_call(
        body,
        grid=(_N1 // _BM,),
        in_specs=[
            pl.BlockSpec((_BM, _D_IN), lambda r: (r, 0)),
            pl.BlockSpec((_D_IN, _H), lambda r: (0, 0)),
            pl.BlockSpec((1, 32, 2048), lambda r: (r, 0, 0)),
        ],
        out_specs=pl.BlockSpec((_BM, _H), lambda r: (r, 0)),
        out_shape=jax.ShapeDtypeStruct((_N1, _H), jnp.float32),
    )(x, W1, deg)


def _table2_tc(p0, p1, b1, W2, deg):
    NB = _N1 // _BM

    def body(p0_ref, p1_ref, b_ref, w_ref, dn_ref, ds_ref, o_ref):
        nd1 = _norm_cols(dn_ref[...])
        ns2 = _norm_cols(ds_ref[...])
        h = (p0_ref[0] + p1_ref[0]) * nd1 + b_ref[...]
        h = _leaky(h)
        o_ref[...] = jnp.dot(h, w_ref[...],
                             preferred_element_type=jnp.float32) * ns2

    return pl.pallas_call(
        body,
        grid=(2, NB),
        in_specs=[
            pl.BlockSpec((1, _BM, _H), lambda c, r: (0, r, 0)),
            pl.BlockSpec((1, _BM, _H), lambda c, r: (1, r, 0)),
            pl.BlockSpec((1, _H), lambda c, r: (0, 0)),
            pl.BlockSpec((_H, _H), lambda c, r: (0, 0)),
            pl.BlockSpec((1, 32, 2048), lambda c, r: (NB + r, 0, 0)),
            pl.BlockSpec((1, 32, 2048), lambda c, r: (2 * NB + c * NB + r, 0, 0)),
        ],
        out_specs=pl.BlockSpec((_BM, _H), lambda c, r: (c * NB + r, 0)),
        out_shape=jax.ShapeDtypeStruct((_N2, _H), jnp.float32),
    )(p0, p1, b1, W2, deg, deg)


def _final_tc(p0, p1, b2, deg):
    NB1 = _N1 // _BM

    def body(p0_ref, p1_ref, b_ref, d_ref, o_ref):
        nd2 = _norm_cols(d_ref[...])
        o_ref[...] = _leaky((p0_ref[0] + p1_ref[0]) * nd2 + b_ref[...])

    return pl.pallas_call(
        body,
        grid=(_N2 // _BM,),
        in_specs=[
            pl.BlockSpec((1, _BM, _H), lambda r: (0, r, 0)),
            pl.BlockSpec((1, _BM, _H), lambda r: (1, r, 0)),
            pl.BlockSpec((1, _H), lambda r: (0, 0)),
            pl.BlockSpec((1, 32, 2048), lambda r: (4 * NB1 + r, 0, 0)),
        ],
        out_specs=pl.BlockSpec((_BM, _H), lambda r: (r, 0)),
        out_shape=jax.ShapeDtypeStruct((_N2, _H), jnp.float32),
    )(p0, p1, b2, deg)


_hist = _make_hist()
_edge1 = _make_edge_pass(_N1, _E1)
_edge2 = _make_edge_pass(_N2, _E2)


def kernel(node_features, edge_index1, edge_index2, W1, b1, W2, b2):
    e1h = edge_index1.reshape(2, 32, _E1 // 32)
    e2h = edge_index2.reshape(2, 32, _E2 // 32)
    e1c = edge_index1.reshape(2, 32, _E1 // _C // 32, _C)
    e2c = edge_index2.reshape(2, 32, _E2 // _C // 32, _C)

    deg = _hist(e1h, e2h)

    table1 = _table1_tc(node_features, W1, deg)
    p1 = _edge1(table1, e1c)
    table2 = _table2_tc(p1, p1, b1[None, :], W2, deg)
    p2 = _edge2(table2, e2c)
    return _final_tc(p2, p2, b2[None, :], deg)


# speedup vs baseline: 1.2780x; 1.0016x over previous
import functools
import jax
import jax.numpy as jnp
from jax import lax
from jax.experimental import pallas as pl
from jax.experimental.pallas import tpu as pltpu
from jax.experimental.pallas import tpu_sc as plsc

_N1 = 10000
_N2 = 20000
_E1 = 320000
_E2 = 640000
_D_IN = 128
_H = 64
_C = 125
_NEG = 0.01
_HTOT = 2 * _N1 + 2 * _N2


def _leaky(x):
    return jnp.where(x >= 0.0, x, x * _NEG)


def _make_hist():
    mesh = plsc.VectorSubcoreMesh(core_axis_name="c", subcore_axis_name="s")
    epts = [_E1 // 32, _E1 // 32, _E2 // 32, _E2 // 32]
    offs = [0, _N1, 2 * _N1, 2 * _N1 + _N2]

    @functools.partial(
        pl.kernel,
        out_type=jax.ShapeDtypeStruct((_HTOT // 2000, 32, 2048), jnp.int32),
        mesh=mesh,
        scratch_types=[
            pltpu.VMEM((_HTOT,), jnp.int32),
            pltpu.VMEM((_E2 // 32,), jnp.int32),
            pltpu.SemaphoreType.DMA,
        ],
        compiler_params=pltpu.CompilerParams(needs_layout_passes=False,
                                             use_tc_tiling_on_sc=False),
    )
    def hist_kernel(e1, e2, out, hist, ibuf, osem):
        cid = lax.axis_index("c")
        sid = lax.axis_index("s")
        wid = cid * 16 + sid
        zero16 = jnp.zeros((16,), jnp.int32)

        def zloop(j, carry):
            hist[pl.ds(j * 16, 16)] = zero16
            return carry

        lax.fori_loop(0, _HTOT // 16, zloop, 0)

        one16 = jnp.ones((16,), jnp.int32)
        srcs = [(e1, 0), (e1, 1), (e2, 0), (e2, 1)]
        for (arr, row), ept, off in zip(srcs, epts, offs):
            pltpu.sync_copy(arr.at[row, wid], ibuf.at[pl.ds(0, ept)])

            def body(j, carry, _off=off):
                v = ibuf[pl.ds(j * 16, 16)] + _off
                plsc.addupdate_scatter(hist, [v], one16)
                return carry

            lax.fori_loop(0, ept // 16, body, 0)

        for k in range(_HTOT // 2000):
            pltpu.async_copy(hist.at[pl.ds(k * 2000, 2000)],
                             out.at[k, wid, pl.ds(0, 2000)], osem)
        for k in range(_HTOT // 2000):
            pltpu.make_async_copy(hist.at[pl.ds(k * 2000, 2000)],
                                  out.at[k, wid, pl.ds(0, 2000)], osem).wait()

    return hist_kernel


def _make_edge_pass(N, E):
    NCH = E // _C
    NCH_T = NCH // 32
    GB = 40
    NG = NCH_T // GB
    NB = 5
    WT = 10
    ROWS_W = N // WT
    ZB = 40
    NZ = ROWS_W // ZB
    mesh = plsc.VectorSubcoreMesh(core_axis_name="c", subcore_axis_name="s")

    @functools.partial(
        pl.kernel,
        out_type=jax.ShapeDtypeStruct((2, N, _H), jnp.float32),
        mesh=mesh,
        scratch_types=[
            pltpu.VMEM_SHARED((N, _H), jnp.float32),
            pltpu.VMEM((GB, _C), jnp.int32),
            pltpu.VMEM((GB, _C), jnp.int32),
            [pltpu.VMEM((_C, _H), jnp.float32)] * NB,
            [pltpu.SemaphoreType.DMA] * NB,
            [pltpu.SemaphoreType.DMA] * NB,
        ],
        compiler_params=pltpu.CompilerParams(needs_layout_passes=False,
                                             use_tc_tiling_on_sc=False),
    )
    def ep(table, eidx, out, acc, sidx, didx, rows, gsem, ssem):
        cid = lax.axis_index("c")
        sid = lax.axis_index("s")
        wid = cid * 16 + sid
        zero16 = jnp.zeros((16,), jnp.float32)

        def zrow(r, carry):
            for c4 in range(_H // 16):
                rows[0][r, pl.ds(c4 * 16, 16)] = zero16
            return carry

        lax.fori_loop(0, ZB, zrow, 0)

        rbase = sid * ROWS_W

        @pl.when(sid < WT)
        def _():
            for k in range(NZ):
                pltpu.sync_copy(rows[0].at[pl.ds(0, ZB)],
                                acc.at[pl.ds(rbase + k * ZB, ZB)])

        plsc.subcore_barrier()

        def gissue(i, b):
            pltpu.async_copy(table.at[sidx.at[i]], rows[b], gsem[b])

        def gwait(i, b):
            pltpu.make_async_copy(table.at[sidx.at[i]], rows[b],
                                  gsem[b]).wait()

        def sissue(i, b):
            pltpu.async_copy(rows[b], acc.at[didx.at[i]], ssem[b], add=True)

        def swait(i, b):
            pltpu.make_async_copy(rows[b], acc.at[didx.at[i]],
                                  ssem[b]).wait()

        for g in range(NG):
            pltpu.sync_copy(eidx.at[0, wid, pl.ds(g * GB, GB)], sidx)
            pltpu.sync_copy(eidx.at[1, wid, pl.ds(g * GB, GB)], didx)
            for b in range(3):
                gissue(b, b)

            def blk_body(blk, carry):
                for j in range(NB):
                    i = blk * NB + j
                    bf = (j + 3) % NB

                    @pl.when(i >= 2)
                    def _():
                        swait(i - 2, bf)

                    @pl.when(i + 3 < GB)
                    def _():
                        gissue(i + 3, bf)

                    gwait(i, j)
                    sissue(i, j)
                return carry

            lax.fori_loop(0, GB // NB, blk_body, 0)
            swait(GB - 2, (GB - 2) % NB)
            swait(GB - 1, (GB - 1) % NB)

        plsc.subcore_barrier()

        @pl.when(sid < WT)
        def _():
            pltpu.sync_copy(acc.at[pl.ds(rbase, ROWS_W)],
                            out.at[cid, pl.ds(rbase, ROWS_W)])

    return ep


_BM = 2000


def _norm_cols(deg_blk):
    s = jnp.sum(deg_blk[0], axis=0)[:_BM][:, None].astype(jnp.float32)
    return lax.rsqrt(jnp.maximum(s, 1.0))


def _table1_tc(x, W1, deg):
    def body(x_ref, w_ref, d_ref, o_ref):
        ns1 = _norm_cols(d_ref[...])
        o_ref[...] = jnp.dot(x_ref[...], w_ref[...],
                             preferred_element_type=jnp.float32) * ns1

    return pl.pallas_call(
        body,
        grid=(_N1 // _BM,),
        in_specs=[
            pl.BlockSpec((_BM, _D_IN), lambda r: (r, 0)),
            pl.BlockSpec((_D_IN, _H), lambda r: (0, 0)),
            pl.BlockSpec((1, 32, 2048), lambda r: (r, 0, 0)),
        ],
        out_specs=pl.BlockSpec((_BM, _H), lambda r: (r, 0)),
        out_shape=jax.ShapeDtypeStruct((_N1, _H), jnp.float32),
    )(x, W1, deg)


def _table2_tc(p0, p1, b1, W2, deg):
    NB = _N1 // _BM

    def body(p0_ref, p1_ref, b_ref, w_ref, dn_ref, ds_ref, o_ref):
        nd1 = _norm_cols(dn_ref[...])
        ns2 = _norm_cols(ds_ref[...])
        h = (p0_ref[0] + p1_ref[0]) * nd1 + b_ref[...]
        h = _leaky(h)
        o_ref[...] = jnp.dot(h, w_ref[...],
                             preferred_element_type=jnp.float32) * ns2

    return pl.pallas_call(
        body,
        grid=(2, NB),
        in_specs=[
            pl.BlockSpec((1, _BM, _H), lambda c, r: (0, r, 0)),
            pl.BlockSpec((1, _BM, _H), lambda c, r: (1, r, 0)),
            pl.BlockSpec((1, _H), lambda c, r: (0, 0)),
            pl.BlockSpec((_H, _H), lambda c, r: (0, 0)),
            pl.BlockSpec((1, 32, 2048), lambda c, r: (NB + r, 0, 0)),
            pl.BlockSpec((1, 32, 2048), lambda c, r: (2 * NB + c * NB + r, 0, 0)),
        ],
        out_specs=pl.BlockSpec((_BM, _H), lambda c, r: (c * NB + r, 0)),
        out_shape=jax.ShapeDtypeStruct((_N2, _H), jnp.float32),
    )(p0, p1, b1, W2, deg, deg)


def _final_tc(p0, p1, b2, deg):
    NB1 = _N1 // _BM

    def body(p0_ref, p1_ref, b_ref, d_ref, o_ref):
        nd2 = _norm_cols(d_ref[...])
        o_ref[...] = _leaky((p0_ref[0] + p1_ref[0]) * nd2 + b_ref[...])

    return pl.pallas_call(
        body,
        grid=(_N2 // _BM,),
        in_specs=[
            pl.BlockSpec((1, _BM, _H), lambda r: (0, r, 0)),
            pl.BlockSpec((1, _BM, _H), lambda r: (1, r, 0)),
            pl.BlockSpec((1, _H), lambda r: (0, 0)),
            pl.BlockSpec((1, 32, 2048), lambda r: (4 * NB1 + r, 0, 0)),
        ],
        out_specs=pl.BlockSpec((_BM, _H), lambda r: (r, 0)),
        out_shape=jax.ShapeDtypeStruct((_N2, _H), jnp.float32),
    )(p0, p1, b2, deg)


_hist = _make_hist()
_edge1 = _make_edge_pass(_N1, _E1)
_edge2 = _make_edge_pass(_N2, _E2)


def kernel(node_features, edge_index1, edge_index2, W1, b1, W2, b2):
    e1h = edge_index1.reshape(2, 32, _E1 // 32)
    e2h = edge_index2.reshape(2, 32, _E2 // 32)
    e1c = edge_index1.reshape(2, 32, _E1 // _C // 32, _C)
    e2c = edge_index2.reshape(2, 32, _E2 // _C // 32, _C)

    deg = _hist(e1h, e2h)

    table1 = _table1_tc(node_features, W1, deg)
    p1 = _edge1(table1, e1c)
    table2 = _table2_tc(p1, p1, b1[None, :], W2, deg)
    p2 = _edge2(table2, e2c)
    return _final_tc(p2, p2, b2[None, :], deg)


# speedup vs baseline: 1.3735x; 1.0747x over previous
import functools
import jax
import jax.numpy as jnp
from jax import lax
from jax.experimental import pallas as pl
from jax.experimental.pallas import tpu as pltpu
from jax.experimental.pallas import tpu_sc as plsc

_N1 = 10000
_N2 = 20000
_E1 = 320000
_E2 = 640000
_D_IN = 128
_H = 64
_C = 125
_NEG = 0.01
_HTOT = 2 * _N1 + 2 * _N2


def _leaky(x):
    return jnp.where(x >= 0.0, x, x * _NEG)


def _make_hist():
    mesh = plsc.VectorSubcoreMesh(core_axis_name="c", subcore_axis_name="s")
    epts = [_E1 // 32, _E1 // 32, _E2 // 32, _E2 // 32]
    offs = [0, _N1, 2 * _N1, 2 * _N1 + _N2]

    @functools.partial(
        pl.kernel,
        out_type=jax.ShapeDtypeStruct((_HTOT // 2000, 32, 2048), jnp.int32),
        mesh=mesh,
        scratch_types=[
            pltpu.VMEM((_HTOT,), jnp.int32),
            pltpu.VMEM((_E2 // 32,), jnp.int32),
            pltpu.SemaphoreType.DMA,
        ],
        compiler_params=pltpu.CompilerParams(needs_layout_passes=False,
                                             use_tc_tiling_on_sc=False),
    )
    def hist_kernel(e1, e2, out, hist, ibuf, osem):
        cid = lax.axis_index("c")
        sid = lax.axis_index("s")
        wid = cid * 16 + sid
        zero16 = jnp.zeros((16,), jnp.int32)

        def zloop(j, carry):
            hist[pl.ds(j * 16, 16)] = zero16
            return carry

        lax.fori_loop(0, _HTOT // 16, zloop, 0)

        one16 = jnp.ones((16,), jnp.int32)
        srcs = [(e1, 0), (e1, 1), (e2, 0), (e2, 1)]
        for (arr, row), ept, off in zip(srcs, epts, offs):
            pltpu.sync_copy(arr.at[row, wid], ibuf.at[pl.ds(0, ept)])

            def body(j, carry, _off=off):
                v = ibuf[pl.ds(j * 16, 16)] + _off
                plsc.addupdate_scatter(hist, [v], one16)
                return carry

            lax.fori_loop(0, ept // 16, body, 0)

        for k in range(_HTOT // 2000):
            pltpu.async_copy(hist.at[pl.ds(k * 2000, 2000)],
                             out.at[k, wid, pl.ds(0, 2000)], osem)
        for k in range(_HTOT // 2000):
            pltpu.make_async_copy(hist.at[pl.ds(k * 2000, 2000)],
                                  out.at[k, wid, pl.ds(0, 2000)], osem).wait()

    return hist_kernel


def _make_edge_pass(N, E):
    NCH = E // _C
    NCH_T = NCH // 32
    GB = 40
    NG = NCH_T // GB
    NB = 5
    WT = 10
    ROWS_W = N // WT
    ZB = 40
    NZ = ROWS_W // ZB
    mesh = plsc.VectorSubcoreMesh(core_axis_name="c", subcore_axis_name="s")

    @functools.partial(
        pl.kernel,
        out_type=jax.ShapeDtypeStruct((2, N, 2 * _H), jnp.float32),
        mesh=mesh,
        scratch_types=[
            pltpu.VMEM_SHARED((N, _H), jnp.float32),
            pltpu.VMEM((GB, _C), jnp.int32),
            pltpu.VMEM((GB, _C), jnp.int32),
            [pltpu.VMEM((_C, _H), jnp.float32)] * NB,
            [pltpu.SemaphoreType.DMA] * NB,
            [pltpu.SemaphoreType.DMA] * NB,
        ],
        compiler_params=pltpu.CompilerParams(needs_layout_passes=False,
                                             use_tc_tiling_on_sc=False),
    )
    def ep(table, eidx, out, acc, sidx, didx, rows, gsem, ssem):
        cid = lax.axis_index("c")
        sid = lax.axis_index("s")
        wid = cid * 16 + sid
        zero16 = jnp.zeros((16,), jnp.float32)

        def zrow(r, carry):
            for c4 in range(_H // 16):
                rows[0][r, pl.ds(c4 * 16, 16)] = zero16
            return carry

        lax.fori_loop(0, ZB, zrow, 0)

        rbase = sid * ROWS_W

        @pl.when(sid < WT)
        def _():
            for k in range(NZ):
                pltpu.sync_copy(rows[0].at[pl.ds(0, ZB)],
                                acc.at[pl.ds(rbase + k * ZB, ZB)])

        plsc.subcore_barrier()

        def gissue(i, b):
            pltpu.async_copy(table.at[sidx.at[i]], rows[b], gsem[b])

        def gwait(i, b):
            pltpu.make_async_copy(table.at[sidx.at[i]], rows[b],
                                  gsem[b]).wait()

        def sissue(i, b):
            pltpu.async_copy(rows[b], acc.at[didx.at[i]], ssem[b], add=True)

        def swait(i, b):
            pltpu.make_async_copy(rows[b], acc.at[didx.at[i]],
                                  ssem[b]).wait()

        for g in range(NG):
            pltpu.sync_copy(eidx.at[0, wid, pl.ds(g * GB, GB)], sidx)
            pltpu.sync_copy(eidx.at[1, wid, pl.ds(g * GB, GB)], didx)
            for b in range(3):
                gissue(b, b)

            def blk_body(blk, carry):
                for j in range(NB):
                    i = blk * NB + j
                    bf = (j + 3) % NB

                    @pl.when(i >= 2)
                    def _():
                        swait(i - 2, bf)

                    @pl.when(i + 3 < GB)
                    def _():
                        gissue(i + 3, bf)

                    gwait(i, j)
                    sissue(i, j)
                return carry

            lax.fori_loop(0, GB // NB, blk_body, 0)
            swait(GB - 2, (GB - 2) % NB)
            swait(GB - 1, (GB - 1) % NB)

        plsc.subcore_barrier()

        @pl.when(sid < WT)
        def _():
            pltpu.sync_copy(acc.at[pl.ds(rbase, ROWS_W)],
                            out.at[cid, pl.ds(rbase, ROWS_W), pl.ds(0, _H)])

    return ep


_BM = 2000


def _norm_cols(deg_blk):
    s = jnp.sum(deg_blk[0], axis=0)[:_BM][:, None].astype(jnp.float32)
    return lax.rsqrt(jnp.maximum(s, 1.0))


def _table1_tc(x, W1, deg):
    def body(x_ref, w_ref, d_ref, o_ref):
        ns1 = _norm_cols(d_ref[...])
        o_ref[...] = jnp.dot(x_ref[...], w_ref[...],
                             preferred_element_type=jnp.float32) * ns1

    return pl.pallas_call(
        body,
        grid=(_N1 // _BM,),
        in_specs=[
            pl.BlockSpec((_BM, _D_IN), lambda r: (r, 0)),
            pl.BlockSpec((_D_IN, _H), lambda r: (0, 0)),
            pl.BlockSpec((1, 32, 2048), lambda r: (r, 0, 0)),
        ],
        out_specs=pl.BlockSpec((_BM, _H), lambda r: (r, 0)),
        out_shape=jax.ShapeDtypeStruct((_N1, _H), jnp.float32),
    )(x, W1, deg)


def _table2_tc(p0, p1, b1, W2, deg):
    NB = _N1 // _BM

    def body(p0_ref, p1_ref, b_ref, w_ref, dn_ref, ds_ref, o_ref):
        nd1 = _norm_cols(dn_ref[...])
        ns2 = _norm_cols(ds_ref[...])
        pp = p0_ref[0][:, :_H] + p1_ref[0][:, :_H]
        h = pp * nd1 + b_ref[...]
        h = _leaky(h)
        o_ref[...] = jnp.dot(h, w_ref[...],
                             preferred_element_type=jnp.float32) * ns2

    return pl.pallas_call(
        body,
        grid=(2, NB),
        in_specs=[
            pl.BlockSpec((1, _BM, 2 * _H), lambda c, r: (0, r, 0)),
            pl.BlockSpec((1, _BM, 2 * _H), lambda c, r: (1, r, 0)),
            pl.BlockSpec((1, _H), lambda c, r: (0, 0)),
            pl.BlockSpec((_H, _H), lambda c, r: (0, 0)),
            pl.BlockSpec((1, 32, 2048), lambda c, r: (NB + r, 0, 0)),
            pl.BlockSpec((1, 32, 2048), lambda c, r: (2 * NB + c * NB + r, 0, 0)),
        ],
        out_specs=pl.BlockSpec((_BM, _H), lambda c, r: (c * NB + r, 0)),
        out_shape=jax.ShapeDtypeStruct((_N2, _H), jnp.float32),
    )(p0, p1, b1, W2, deg, deg)


def _final_tc(p0, p1, b2, deg):
    NB1 = _N1 // _BM

    def body(p0_ref, p1_ref, b_ref, d_ref, o_ref):
        nd2 = _norm_cols(d_ref[...])
        pp = p0_ref[0][:, :_H] + p1_ref[0][:, :_H]
        o_ref[...] = _leaky(pp * nd2 + b_ref[...])

    return pl.pallas_call(
        body,
        grid=(_N2 // _BM,),
        in_specs=[
            pl.BlockSpec((1, _BM, 2 * _H), lambda r: (0, r, 0)),
            pl.BlockSpec((1, _BM, 2 * _H), lambda r: (1, r, 0)),
            pl.BlockSpec((1, _H), lambda r: (0, 0)),
            pl.BlockSpec((1, 32, 2048), lambda r: (4 * NB1 + r, 0, 0)),
        ],
        out_specs=pl.BlockSpec((_BM, _H), lambda r: (r, 0)),
        out_shape=jax.ShapeDtypeStruct((_N2, _H), jnp.float32),
    )(p0, p1, b2, deg)


_hist = _make_hist()
_edge1 = _make_edge_pass(_N1, _E1)
_edge2 = _make_edge_pass(_N2, _E2)


def kernel(node_features, edge_index1, edge_index2, W1, b1, W2, b2):
    e1h = edge_index1.reshape(2, 32, _E1 // 32)
    e2h = edge_index2.reshape(2, 32, _E2 // 32)
    e1c = edge_index1.reshape(2, 32, _E1 // _C // 32, _C)
    e2c = edge_index2.reshape(2, 32, _E2 // _C // 32, _C)

    deg = _hist(e1h, e2h)

    table1 = _table1_tc(node_features, W1, deg)
    p1 = _edge1(table1, e1c)
    table2 = _table2_tc(p1, p1, b1[None, :], W2, deg)
    p2 = _edge2(table2, e2c)
    return _final_tc(p2, p2, b2[None, :], deg)
